# baseline (speedup 1.0000x reference)
import functools

import jax
import jax.numpy as jnp
from jax import lax
from jax.experimental import pallas as pl
from jax.experimental.pallas import tpu as pltpu
from jax.experimental.pallas import tpu_sc as plsc

N = 25000
D = 128
T = 4
BASE1 = 12544
ACC_ROWS = 13056
K = 64
NB = 3
NS = 16
ROW_BLK = 5000


def _linear_tc(x, w, b):
    n, din = x.shape
    dout = w.shape[1]

    def body(x_ref, w_ref, b_ref, o_ref):
        o_ref[...] = jnp.dot(x_ref[...], w_ref[...],
                             preferred_element_type=jnp.float32) + b_ref[...]

    return pl.pallas_call(
        body,
        grid=(n // ROW_BLK,),
        in_specs=[pl.BlockSpec((ROW_BLK, din), lambda i: (i, 0)),
                  pl.BlockSpec((din, dout), lambda i: (0, 0)),
                  pl.BlockSpec((1, dout), lambda i: (0, 0))],
        out_specs=pl.BlockSpec((ROW_BLK, dout), lambda i: (i, 0)),
        out_shape=jax.ShapeDtypeStruct((n, dout), jnp.float32),
    )(x, w, b.reshape(1, dout))


def _mlp2_tc(x, w0, b0, w1, b1):
    n, din = x.shape
    dh = w0.shape[1]
    dout = w1.shape[1]

    def body(x_ref, w0_ref, b0_ref, w1_ref, b1_ref, o_ref):
        h = jnp.tanh(jnp.dot(x_ref[...], w0_ref[...],
                             preferred_element_type=jnp.float32) + b0_ref[...])
        o_ref[...] = jnp.dot(h, w1_ref[...],
                             preferred_element_type=jnp.float32) + b1_ref[...]

    return pl.pallas_call(
        body,
        grid=(n // ROW_BLK,),
        in_specs=[pl.BlockSpec((ROW_BLK, din), lambda i: (i, 0)),
                  pl.BlockSpec((din, dh), lambda i: (0, 0)),
                  pl.BlockSpec((1, dh), lambda i: (0, 0)),
                  pl.BlockSpec((dh, dout), lambda i: (0, 0)),
                  pl.BlockSpec((1, dout), lambda i: (0, 0))],
        out_specs=pl.BlockSpec((ROW_BLK, dout), lambda i: (i, 0)),
        out_shape=jax.ShapeDtypeStruct((n, dout), jnp.float32),
    )(x, w0, b0.reshape(1, dh), w1, b1.reshape(1, dout))


def _upd_tc(agg, h, w0, b0, w1, b1):
    n = agg.shape[0]

    def body(a_ref, h_ref, w0_ref, b0_ref, w1_ref, b1_ref, o_ref):
        x = jnp.concatenate([a_ref[...], h_ref[...]], axis=1)
        t = jnp.tanh(jnp.dot(x, w0_ref[...],
                             preferred_element_type=jnp.float32) + b0_ref[...])
        o_ref[...] = jnp.dot(t, w1_ref[...],
                             preferred_element_type=jnp.float32) + b1_ref[...]

    return pl.pallas_call(
        body,
        grid=(n // ROW_BLK,),
        in_specs=[pl.BlockSpec((ROW_BLK, D), lambda i: (i, 0)),
                  pl.BlockSpec((ROW_BLK, D), lambda i: (i, 0)),
                  pl.BlockSpec((2 * D, D), lambda i: (0, 0)),
                  pl.BlockSpec((1, D), lambda i: (0, 0)),
                  pl.BlockSpec((D, D), lambda i: (0, 0)),
                  pl.BlockSpec((1, D), lambda i: (0, 0))],
        out_specs=pl.BlockSpec((ROW_BLK, D), lambda i: (i, 0)),
        out_shape=jax.ShapeDtypeStruct((n, D), jnp.float32),
    )(agg, h, w0, b0.reshape(1, D), w1, b1.reshape(1, D))


def _bcast16(vec, lane):
    idx = jnp.full((16, 1), lane, jnp.int32)
    return lax.gather(
        vec, idx,
        dimension_numbers=lax.GatherDimensionNumbers(
            offset_dims=(), collapsed_slice_dims=(0,), start_index_map=(0,)),
        slice_sizes=(1,),
        mode=lax.GatherScatterMode.PROMISE_IN_BOUNDS)

def _make_seg_agg(e_pad):
    epw = e_pad // NS
    n_it = epw // K
    mesh = plsc.VectorSubcoreMesh(core_axis_name="c", subcore_axis_name="s")

    @functools.partial(
        pl.kernel,
        out_type=jax.ShapeDtypeStruct((N, D), jnp.float32),
        mesh=mesh,
        scratch_types=(
            [pltpu.VMEM_SHARED((ACC_ROWS, D), jnp.float32)]
            + [pltpu.VMEM((K,), jnp.int32) for _ in range(NB)]
            + [pltpu.VMEM((K,), jnp.int32) for _ in range(NB)]
            + [pltpu.VMEM((K,), jnp.int32) for _ in range(NB)]
            + [pltpu.VMEM((K,), jnp.float32) for _ in range(NB)]
            + [pltpu.VMEM((K, D), jnp.float32) for _ in range(NB)]
            + [pltpu.SemaphoreType.DMA for _ in range(3 * NB)]
        ),
    )
    def seg_agg(m_hbm, src_hbm, dst_hbm, vals_hbm, out_hbm, acc, *scr):
        src_v = scr[0:NB]
        dst_v = scr[NB:2 * NB]
        ldst_v = scr[2 * NB:3 * NB]
        vals_v = scr[3 * NB:4 * NB]
        rows_v = scr[4 * NB:5 * NB]
        sem_i = scr[5 * NB:6 * NB]
        sem_g = scr[6 * NB:7 * NB]
        sem_s = scr[7 * NB:8 * NB]
        c = lax.axis_index("c")
        s = lax.axis_index("s")

        def zrow(j, _):
            for d in range(D // 16):
                rows_v[0][j, pl.ds(d * 16, 16)] = jnp.zeros((16,), jnp.float32)
            return 0
        lax.fori_loop(0, K, zrow, 0)
        zbase = s * (ACC_ROWS // NS)
        for kchunk in range(12):
            pltpu.sync_copy(rows_v[0], acc.at[pl.ds(zbase + kchunk * K, K)])
        pltpu.sync_copy(rows_v[0].at[pl.ds(0, 48)],
                        acc.at[pl.ds(zbase + 12 * K, 48)])
        plsc.subcore_barrier()

        base = s * epw
        n_own = jnp.where(c == 0, BASE1, N - BASE1)

        def idx_issue(i, b):
            off = base + i * K
            pltpu.async_copy(src_hbm.at[pl.ds(off, K)], src_v[b], sem_i[b])
            pltpu.async_copy(dst_hbm.at[pl.ds(off, K)], dst_v[b], sem_i[b])
            pltpu.async_copy(vals_hbm.at[pl.ds(off, K)], vals_v[b], sem_i[b])

        def idx_wait(i, b):
            off = base + i * K
            pltpu.make_async_copy(src_hbm.at[pl.ds(off, K)], src_v[b],
                                  sem_i[b]).wait()
            pltpu.make_async_copy(dst_hbm.at[pl.ds(off, K)], dst_v[b],
                                  sem_i[b]).wait()
            pltpu.make_async_copy(vals_hbm.at[pl.ds(off, K)], vals_v[b],
                                  sem_i[b]).wait()

        def gather_issue(b):
            pltpu.async_copy(m_hbm.at[src_v[b]], rows_v[b], sem_g[b])

        def gather_wait(b):
            pltpu.make_async_copy(m_hbm.at[src_v[b]], rows_v[b],
                                  sem_g[b]).wait()

        def scatter_issue(b):
            pltpu.async_copy(rows_v[b], acc.at[ldst_v[b]], sem_s[b], add=True)

        def scatter_wait(b):
            pltpu.make_async_copy(rows_v[b], acc.at[ldst_v[b]],
                                  sem_s[b]).wait()

        for b in range(NB):
            idx_issue(b, b)
        for b in range(2):
            idx_wait(b, b)
            gather_issue(b)

        def group_body(g, _):
            for bb in range(NB):
                i = g * NB + bb
                b2 = (bb + 2) % NB

                @pl.when(i > 0)
                def _():
                    scatter_wait(b2)
                gather_wait(bb)
                for t in range(K // 16):
                    dv = dst_v[bb][pl.ds(t * 16, 16)]
                    lv = dv - c * BASE1
                    inr = (lv >= 0) & (lv < n_own)
                    dummy = BASE1 + (dv & 511)
                    ldst_v[bb][pl.ds(t * 16, 16)] = jnp.where(inr, lv, dummy)

                for gi in range(K // 16):
                    vv = vals_v[bb][pl.ds(gi * 16, 16)]
                    for l in range(16):
                        sv = _bcast16(vv, l)
                        for d in range(D // 16):
                            sl = (gi * 16 + l, pl.ds(d * 16, 16))
                            rows_v[bb][sl] = rows_v[bb][sl] * sv

                scatter_issue(bb)

                @pl.when(i + 3 < n_it)
                def _():
                    idx_issue(i + 3, bb)

                @pl.when(i + 2 < n_it)
                def _():
                    idx_wait(i + 2, b2)
                    gather_issue(b2)
            return 0

        lax.fori_loop(0, n_it // NB, group_body, 0)
        scatter_wait((n_it - 1) % NB)
        plsc.subcore_barrier()

        @pl.when((s < NS - 1) | (c == 0))
        def _():
            pltpu.sync_copy(acc.at[pl.ds(s * 784, 784)],
                            out_hbm.at[pl.ds(c * BASE1 + s * 784, 784)])

        @pl.when((s == NS - 1) & (c == 1))
        def _():
            pltpu.sync_copy(acc.at[pl.ds((NS - 1) * 784, 696)],
                            out_hbm.at[pl.ds(BASE1 + (NS - 1) * 784, 696)])

    return seg_agg


def kernel(x_pv, x_pq, edge_index, edge_vals, params):
    p = params
    rows = edge_index[0].astype(jnp.int32)
    cols = edge_index[1].astype(jnp.int32)
    e = rows.shape[0]
    e_pad = ((e + NS * K * NB - 1) // (NS * K * NB)) * (NS * K * NB)
    pad_n = e_pad - e
    if pad_n:
        fill = (jnp.arange(pad_n, dtype=jnp.int32) * 97) % N
        rows = jnp.concatenate([rows, fill])
        cols = jnp.concatenate([cols, fill])
        vals = jnp.concatenate([edge_vals, jnp.zeros((pad_n,), jnp.float32)])
    else:
        vals = edge_vals

    seg_agg = _make_seg_agg(e_pad)

    h_pv = _linear_tc(x_pv, p['enc_pv_w'], p['enc_pv_b'])
    h_pq = _linear_tc(x_pq, p['enc_pq_w'], p['enc_pq_b'])

    for _ in range(T):
        m_pq = _mlp2_tc(h_pq, p['msg_pq2pv_w0'], p['msg_pq2pv_b0'],
                        p['msg_pq2pv_w1'], p['msg_pq2pv_b1'])
        agg_pv = seg_agg(m_pq, cols, rows, vals)
        m_pv = _mlp2_tc(h_pv, p['msg_pv2pq_w0'], p['msg_pv2pq_b0'],
                        p['msg_pv2pq_w1'], p['msg_pv2pq_b1'])
        agg_pq = seg_agg(m_pv, rows, cols, vals)
        h_pv = _upd_tc(agg_pv, h_pv, p['upd_pv_w0'],
                       p['upd_pv_b0'], p['upd_pv_w1'], p['upd_pv_b1'])
        h_pq = _upd_tc(agg_pq, h_pq, p['upd_pq_w0'],
                       p['upd_pq_b0'], p['upd_pq_w1'], p['upd_pq_b1'])

    out_pv = _linear_tc(h_pv, p['dec_pv_w'], p['dec_pv_b'])
    out_pq = _linear_tc(h_pq, p['dec_pq_w'], p['dec_pq_b'])
    return (out_pv, out_pq)


# speedup vs baseline: 1.3582x; 1.3582x over previous
import functools

import jax
import jax.numpy as jnp
from jax import lax
from jax.experimental import pallas as pl
from jax.experimental.pallas import tpu as pltpu
from jax.experimental.pallas import tpu_sc as plsc

N = 25000
D = 128
T = 4
BASE1 = 12544
ACC_ROWS = 13056
K = 64
NB = 3
NS = 16
ROW_BLK = 5000


def _linear_tc(x, w, b):
    n, din = x.shape
    dout = w.shape[1]

    def body(x_ref, w_ref, b_ref, o_ref):
        o_ref[...] = jnp.dot(x_ref[...], w_ref[...],
                             preferred_element_type=jnp.float32) + b_ref[...]

    return pl.pallas_call(
        body,
        grid=(n // ROW_BLK,),
        in_specs=[pl.BlockSpec((ROW_BLK, din), lambda i: (i, 0)),
                  pl.BlockSpec((din, dout), lambda i: (0, 0)),
                  pl.BlockSpec((1, dout), lambda i: (0, 0))],
        out_specs=pl.BlockSpec((ROW_BLK, dout), lambda i: (i, 0)),
        out_shape=jax.ShapeDtypeStruct((n, dout), jnp.float32),
    )(x, w, b.reshape(1, dout))


def _mlp2_tc(x, w0, b0, w1, b1):
    n, din = x.shape
    dh = w0.shape[1]
    dout = w1.shape[1]

    def body(x_ref, w0_ref, b0_ref, w1_ref, b1_ref, o_ref):
        h = jnp.tanh(jnp.dot(x_ref[...], w0_ref[...],
                             preferred_element_type=jnp.float32) + b0_ref[...])
        o_ref[...] = jnp.dot(h, w1_ref[...],
                             preferred_element_type=jnp.float32) + b1_ref[...]

    return pl.pallas_call(
        body,
        grid=(n // ROW_BLK,),
        in_specs=[pl.BlockSpec((ROW_BLK, din), lambda i: (i, 0)),
                  pl.BlockSpec((din, dh), lambda i: (0, 0)),
                  pl.BlockSpec((1, dh), lambda i: (0, 0)),
                  pl.BlockSpec((dh, dout), lambda i: (0, 0)),
                  pl.BlockSpec((1, dout), lambda i: (0, 0))],
        out_specs=pl.BlockSpec((ROW_BLK, dout), lambda i: (i, 0)),
        out_shape=jax.ShapeDtypeStruct((n, dout), jnp.float32),
    )(x, w0, b0.reshape(1, dh), w1, b1.reshape(1, dout))


def _upd_tc(agg, h, w0, b0, w1, b1):
    n = agg.shape[0]

    def body(a_ref, h_ref, w0_ref, b0_ref, w1_ref, b1_ref, o_ref):
        x = jnp.concatenate([a_ref[...], h_ref[...]], axis=1)
        t = jnp.tanh(jnp.dot(x, w0_ref[...],
                             preferred_element_type=jnp.float32) + b0_ref[...])
        o_ref[...] = jnp.dot(t, w1_ref[...],
                             preferred_element_type=jnp.float32) + b1_ref[...]

    return pl.pallas_call(
        body,
        grid=(n // ROW_BLK,),
        in_specs=[pl.BlockSpec((ROW_BLK, D), lambda i: (i, 0)),
                  pl.BlockSpec((ROW_BLK, D), lambda i: (i, 0)),
                  pl.BlockSpec((2 * D, D), lambda i: (0, 0)),
                  pl.BlockSpec((1, D), lambda i: (0, 0)),
                  pl.BlockSpec((D, D), lambda i: (0, 0)),
                  pl.BlockSpec((1, D), lambda i: (0, 0))],
        out_specs=pl.BlockSpec((ROW_BLK, D), lambda i: (i, 0)),
        out_shape=jax.ShapeDtypeStruct((n, D), jnp.float32),
    )(agg, h, w0, b0.reshape(1, D), w1, b1.reshape(1, D))


def _bcast16(vec, lane):
    idx = jnp.full((16, 1), lane, jnp.int32)
    return lax.gather(
        vec, idx,
        dimension_numbers=lax.GatherDimensionNumbers(
            offset_dims=(), collapsed_slice_dims=(0,), start_index_map=(0,)),
        slice_sizes=(1,),
        mode=lax.GatherScatterMode.PROMISE_IN_BOUNDS)

def _make_seg_agg(e_pad):
    epw = e_pad // NS
    n_it = epw // K
    mesh = plsc.VectorSubcoreMesh(core_axis_name="c", subcore_axis_name="s")

    @functools.partial(
        pl.kernel,
        out_type=jax.ShapeDtypeStruct((N, D), jnp.float32),
        mesh=mesh,
        scratch_types=(
            [pltpu.VMEM_SHARED((ACC_ROWS, D), jnp.float32)]
            + [pltpu.VMEM((K,), jnp.int32) for _ in range(NB)]
            + [pltpu.VMEM((K,), jnp.int32) for _ in range(NB)]
            + [pltpu.VMEM((K,), jnp.int32) for _ in range(NB)]
            + [pltpu.VMEM((K,), jnp.float32) for _ in range(NB)]
            + [pltpu.VMEM((K, D), jnp.float32) for _ in range(NB)]
            + [pltpu.SemaphoreType.DMA for _ in range(3 * NB)]
        ),
    )
    def seg_agg(m_hbm, src_hbm, dst_hbm, vals_hbm, out_hbm, acc, *scr):
        src_v = scr[0:NB]
        dst_v = scr[NB:2 * NB]
        ldst_v = scr[2 * NB:3 * NB]
        vals_v = scr[3 * NB:4 * NB]
        rows_v = scr[4 * NB:5 * NB]
        sem_i = scr[5 * NB:6 * NB]
        sem_g = scr[6 * NB:7 * NB]
        sem_s = scr[7 * NB:8 * NB]
        c = lax.axis_index("c")
        s = lax.axis_index("s")

        def zrow(j, _):
            for d in range(D // 16):
                rows_v[0][j, pl.ds(d * 16, 16)] = jnp.zeros((16,), jnp.float32)
            return 0
        lax.fori_loop(0, K, zrow, 0)
        zbase = s * (ACC_ROWS // NS)
        for kchunk in range(12):
            pltpu.sync_copy(rows_v[0], acc.at[pl.ds(zbase + kchunk * K, K)])
        pltpu.sync_copy(rows_v[0].at[pl.ds(0, 48)],
                        acc.at[pl.ds(zbase + 12 * K, 48)])
        plsc.subcore_barrier()

        base = s * epw
        n_own = jnp.where(c == 0, BASE1, N - BASE1)

        def idx_issue(i, b):
            off = base + i * K
            pltpu.async_copy(src_hbm.at[pl.ds(off, K)], src_v[b], sem_i[b])
            pltpu.async_copy(dst_hbm.at[pl.ds(off, K)], dst_v[b], sem_i[b])
            pltpu.async_copy(vals_hbm.at[pl.ds(off, K)], vals_v[b], sem_i[b])

        def idx_wait(i, b):
            off = base + i * K
            pltpu.make_async_copy(src_hbm.at[pl.ds(off, K)], src_v[b],
                                  sem_i[b]).wait()
            pltpu.make_async_copy(dst_hbm.at[pl.ds(off, K)], dst_v[b],
                                  sem_i[b]).wait()
            pltpu.make_async_copy(vals_hbm.at[pl.ds(off, K)], vals_v[b],
                                  sem_i[b]).wait()

        def gather_issue(b):
            pltpu.async_copy(m_hbm.at[src_v[b]], rows_v[b], sem_g[b])

        def gather_wait(b):
            pltpu.make_async_copy(m_hbm.at[src_v[b]], rows_v[b],
                                  sem_g[b]).wait()

        def scatter_issue(b):
            pltpu.async_copy(rows_v[b], acc.at[ldst_v[b]], sem_s[b], add=True)

        def scatter_wait(b):
            pltpu.make_async_copy(rows_v[b], acc.at[ldst_v[b]],
                                  sem_s[b]).wait()

        for b in range(NB):
            idx_issue(b, b)
        for b in range(2):
            idx_wait(b, b)
            gather_issue(b)

        def group_body(g, _):
            for bb in range(NB):
                i = g * NB + bb
                b2 = (bb + 2) % NB

                @pl.when(i > 0)
                def _():
                    scatter_wait(b2)
                gather_wait(bb)
                for t in range(K // 16):
                    dv = dst_v[bb][pl.ds(t * 16, 16)]
                    lv = dv - c * BASE1
                    inr = (lv >= 0) & (lv < n_own)
                    dummy = BASE1 + (dv & 511)
                    ldst_v[bb][pl.ds(t * 16, 16)] = jnp.where(inr, lv, dummy)

                @plsc.parallel_loop(0, K // 16)
                def _(gi):
                    vv = vals_v[bb][pl.ds(gi * 16, 16)]
                    for l in range(16):
                        sv = _bcast16(vv, l)
                        for d in range(D // 16):
                            sl = (gi * 16 + l, pl.ds(d * 16, 16))
                            rows_v[bb][sl] = rows_v[bb][sl] * sv

                scatter_issue(bb)

                @pl.when(i + 3 < n_it)
                def _():
                    idx_issue(i + 3, bb)

                @pl.when(i + 2 < n_it)
                def _():
                    idx_wait(i + 2, b2)
                    gather_issue(b2)
            return 0

        lax.fori_loop(0, n_it // NB, group_body, 0)
        scatter_wait((n_it - 1) % NB)
        plsc.subcore_barrier()

        @pl.when((s < NS - 1) | (c == 0))
        def _():
            pltpu.sync_copy(acc.at[pl.ds(s * 784, 784)],
                            out_hbm.at[pl.ds(c * BASE1 + s * 784, 784)])

        @pl.when((s == NS - 1) & (c == 1))
        def _():
            pltpu.sync_copy(acc.at[pl.ds((NS - 1) * 784, 696)],
                            out_hbm.at[pl.ds(BASE1 + (NS - 1) * 784, 696)])

    return seg_agg


def kernel(x_pv, x_pq, edge_index, edge_vals, params):
    p = params
    rows = edge_index[0].astype(jnp.int32)
    cols = edge_index[1].astype(jnp.int32)
    e = rows.shape[0]
    e_pad = ((e + NS * K * NB - 1) // (NS * K * NB)) * (NS * K * NB)
    pad_n = e_pad - e
    if pad_n:
        fill = (jnp.arange(pad_n, dtype=jnp.int32) * 97) % N
        rows = jnp.concatenate([rows, fill])
        cols = jnp.concatenate([cols, fill])
        vals = jnp.concatenate([edge_vals, jnp.zeros((pad_n,), jnp.float32)])
    else:
        vals = edge_vals

    seg_agg = _make_seg_agg(e_pad)

    h_pv = _linear_tc(x_pv, p['enc_pv_w'], p['enc_pv_b'])
    h_pq = _linear_tc(x_pq, p['enc_pq_w'], p['enc_pq_b'])

    for _ in range(T):
        m_pq = _mlp2_tc(h_pq, p['msg_pq2pv_w0'], p['msg_pq2pv_b0'],
                        p['msg_pq2pv_w1'], p['msg_pq2pv_b1'])
        agg_pv = seg_agg(m_pq, cols, rows, vals)
        m_pv = _mlp2_tc(h_pv, p['msg_pv2pq_w0'], p['msg_pv2pq_b0'],
                        p['msg_pv2pq_w1'], p['msg_pv2pq_b1'])
        agg_pq = seg_agg(m_pv, rows, cols, vals)
        h_pv = _upd_tc(agg_pv, h_pv, p['upd_pv_w0'],
                       p['upd_pv_b0'], p['upd_pv_w1'], p['upd_pv_b1'])
        h_pq = _upd_tc(agg_pq, h_pq, p['upd_pq_w0'],
                       p['upd_pq_b0'], p['upd_pq_w1'], p['upd_pq_b1'])

    out_pv = _linear_tc(h_pv, p['dec_pv_w'], p['dec_pv_b'])
    out_pq = _linear_tc(h_pq, p['dec_pq_w'], p['dec_pq_b'])
    return (out_pv, out_pq)
